# Initial kernel scaffold; baseline (speedup 1.0000x reference)
import jax
import jax.numpy as jnp
from jax.experimental import pallas as pl


def kernel(tgt_skel_id, z_means, offset):
    raise NotImplementedError("write your pallas kernel here")


# speedup vs baseline: 1.1541x; 1.1541x over previous
import functools

import jax
import jax.numpy as jnp
from jax import lax
from jax.experimental import pallas as pl
from jax.experimental.pallas import tpu as pltpu
from jax.experimental.pallas import tpu_sc as plsc

_N_SKELS = 100000
_N_TRAIN = 80000
_N_TOKENS = 8
_CODE_DIM = 64
_BATCH = 16384
_D = _N_TOKENS * _CODE_DIM

_NC = 2
_NS = 16
_NW = _NC * _NS
_B_PER_W = _BATCH // _NW
_CHUNK = 64
_N_CHUNKS = _B_PER_W // _CHUNK
_LANES = 16


def _make_kernel():
    mesh = plsc.VectorSubcoreMesh(core_axis_name="c", subcore_axis_name="s")

    @functools.partial(
        pl.kernel,
        out_type=jax.ShapeDtypeStruct((_BATCH, _D), jnp.float32),
        mesh=mesh,
        scratch_types=[
            pltpu.VMEM((_B_PER_W,), jnp.int32),
            pltpu.VMEM((_CHUNK, _D), jnp.float32),
            pltpu.VMEM((_CHUNK, _D), jnp.float32),
            pltpu.SemaphoreType.DMA,
            pltpu.SemaphoreType.DMA,
        ],
    )
    def k(idx_hbm, zm_hbm, off_hbm, out_hbm, idx_v, mean_v, off_v, sem_m, sem_o):
        wid = lax.axis_index("s") * _NC + lax.axis_index("c")
        base = wid * _B_PER_W
        pltpu.sync_copy(idx_hbm.at[pl.ds(base, _B_PER_W)], idx_v)

        def chunk_body(ci, carry):
            row0 = ci * _CHUNK
            idx_slice = idx_v.at[pl.ds(row0, _CHUNK)]
            cp_m = pltpu.async_copy(zm_hbm.at[idx_slice], mean_v, sem_m)
            cp_o = pltpu.async_copy(off_hbm.at[idx_slice], off_v, sem_o)
            cp_m.wait()
            cp_o.wait()

            def group_body(g, carry2):
                ids16 = idx_v[pl.ds(row0 + g * _LANES, _LANES)]
                mvec = jnp.where(ids16 < _N_TRAIN, jnp.float32(1.0),
                                 jnp.float32(0.0))

                def row_body(rsub, carry3):
                    lane = (jnp.zeros((_LANES,), jnp.int32) + rsub)[:, None]
                    m_b = lax.gather(
                        mvec, lane,
                        lax.GatherDimensionNumbers(
                            offset_dims=(), collapsed_slice_dims=(0,),
                            start_index_map=(0,)),
                        slice_sizes=(1,),
                        mode=lax.GatherScatterMode.PROMISE_IN_BOUNDS)
                    row = g * _LANES + rsub
                    for c in range(_D // _LANES):
                        sl = pl.ds(c * _LANES, _LANES)
                        mean_v[row, sl] = mean_v[row, sl] + off_v[row, sl] * m_b
                    return carry3

                lax.fori_loop(0, _LANES, row_body, 0, unroll=False)
                return carry2

            lax.fori_loop(0, _CHUNK // _LANES, group_body, 0, unroll=False)
            pltpu.sync_copy(mean_v, out_hbm.at[pl.ds(base + row0, _CHUNK)])
            return carry

        lax.fori_loop(0, _N_CHUNKS, chunk_body, 0, unroll=False)

    return k


_kernel_call = _make_kernel()


@jax.jit
def kernel(tgt_skel_id, z_means, offset):
    zm2 = z_means.reshape(_N_SKELS, _D)
    off2 = offset.reshape(_N_SKELS, _D)
    out = _kernel_call(tgt_skel_id, zm2, off2)
    return out.reshape(_BATCH, _N_TOKENS, _CODE_DIM)
